# baseline (speedup 1.0000x reference)
import math

import jax
import jax.numpy as jnp
from jax.experimental import pallas as pl
from jax.experimental.pallas import tpu as pltpu

_EPS = 1e-3
_CHUNK = 128
_BPS = 2
_LOG2E = math.log2(math.e)


def _ucb_kernel(x_ref, mean_ref, var_ref, out_ref, s_acc, a1_scr, a3_scr):
    ph = pl.program_id(0)
    g = pl.program_id(1)
    n_t, n_d = x_ref.shape[1], x_ref.shape[2]
    n_k = mean_ref.shape[0]
    n_ch = n_t // _CHUNK

    mean = mean_ref[...]
    nc2 = (-0.5 * _LOG2E) / (jax.nn.softplus(var_ref[...]) + _EPS)

    def _vsum(v):
        return jnp.sum(v.reshape(_CHUNK // 8, 8, n_d), axis=0)

    @pl.when(ph == 0)
    def _stats_pass():
        @pl.when(g == 0)
        def _init():
            s_acc[...] = jnp.zeros_like(s_acc)

        for bb in range(_BPS):
            s8 = [jnp.zeros((8, n_d), jnp.float32) for _ in range(n_k)]
            a1 = [jnp.zeros((8, n_d), jnp.float32) for _ in range(n_k)]
            a3 = [jnp.zeros((8, n_d), jnp.float32) for _ in range(n_k)]
            for c in range(n_ch):
                sl = slice(c * _CHUNK, (c + 1) * _CHUNK)
                xc = x_ref[bb, sl, :]
                es = []
                s = None
                for k in range(n_k):
                    d = xc - mean[k : k + 1, :]
                    e = jnp.exp2(d * d * nc2[k : k + 1, :])
                    es.append(e)
                    s = e if s is None else s + e
                r = 1.0 / (s + _EPS)
                for k in range(n_k):
                    tau = es[k] * r
                    t1 = tau * xc
                    t3 = t1 * t1 * tau
                    s8[k] = s8[k] + _vsum(tau)
                    a1[k] = a1[k] + _vsum(t1)
                    a3[k] = a3[k] + _vsum(t3)
            fin = lambda vs: jnp.concatenate(
                [jnp.sum(v, axis=0, keepdims=True) for v in vs], axis=0
            )
            s_acc[...] = s_acc[...] + fin(s8)
            a1_scr[g * _BPS + bb] = fin(a1)
            a3_scr[g * _BPS + bb] = fin(a3)

    @pl.when(ph == 1)
    def _output_pass():
        r_s = 1.0 / (s_acc[...] + _EPS)
        pri_scale = jnp.float32(1.0 / math.sqrt(1.0 + _EPS))
        inv_t = jnp.float32(1.0 / n_t)
        for bb in range(_BPS):
            a1 = a1_scr[g * _BPS + bb]
            a3 = a3_scr[g * _BPS + bb]
            e1 = r_s * a1 * inv_t
            e3 = (r_s * r_s * r_s) * a3 * inv_t
            w = jax.lax.rsqrt(e3 + _EPS) * pri_scale
            cw = e1 * w

            for c in range(n_ch):
                sl = slice(c * _CHUNK, (c + 1) * _CHUNK)
                xc = x_ref[bb, sl, :]
                es = []
                s = None
                for k in range(n_k):
                    d = xc - mean[k : k + 1, :]
                    e = jnp.exp2(d * d * nc2[k : k + 1, :])
                    es.append(e)
                    s = e if s is None else s + e
                r = 1.0 / (s + _EPS)
                p_sum = None
                q_sum = None
                for k in range(n_k):
                    pk = es[k] * w[k : k + 1, :]
                    qk = es[k] * cw[k : k + 1, :]
                    p_sum = pk if p_sum is None else p_sum + pk
                    q_sum = qk if q_sum is None else q_sum + qk
                out_ref[bb, sl, :] = r * (xc * p_sum - q_sum)


def kernel(x, mean, variance, prior):
    del prior
    n_b, n_t, n_d = x.shape
    n_k = mean.shape[0]
    return pl.pallas_call(
        _ucb_kernel,
        grid=(2, n_b // _BPS),
        in_specs=[
            pl.BlockSpec((_BPS, n_t, n_d), lambda ph, g: (g, 0, 0)),
            pl.BlockSpec((n_k, n_d), lambda ph, g: (0, 0)),
            pl.BlockSpec((n_k, n_d), lambda ph, g: (0, 0)),
        ],
        out_specs=pl.BlockSpec((_BPS, n_t, n_d), lambda ph, g: (g * ph, 0, 0)),
        out_shape=jax.ShapeDtypeStruct((n_b, n_t, n_d), jnp.float32),
        scratch_shapes=[
            pltpu.VMEM((n_k, n_d), jnp.float32),
            pltpu.VMEM((n_b, n_k, n_d), jnp.float32),
            pltpu.VMEM((n_b, n_k, n_d), jnp.float32),
        ],
        compiler_params=pltpu.CompilerParams(
            dimension_semantics=("arbitrary", "arbitrary"),
        ),
        name="ucb_norm",
    )(x, mean, variance)


# speedup vs baseline: 1.0011x; 1.0011x over previous
import math

import jax
import jax.numpy as jnp
from jax.experimental import pallas as pl
from jax.experimental.pallas import tpu as pltpu

_EPS = 1e-3
_CHUNK = 128
_BPS = 1
_LOG2E = math.log2(math.e)


def _ucb_kernel(x_ref, mean_ref, var_ref, out_ref, s_acc, a1_scr, a3_scr):
    ph = pl.program_id(0)
    g = pl.program_id(1)
    n_t, n_d = x_ref.shape[1], x_ref.shape[2]
    n_k = mean_ref.shape[0]
    n_ch = n_t // _CHUNK

    mean = mean_ref[...]
    nc2 = (-0.5 * _LOG2E) / (jax.nn.softplus(var_ref[...]) + _EPS)

    def _vsum(v):
        return jnp.sum(v.reshape(_CHUNK // 8, 8, n_d), axis=0)

    @pl.when(ph == 0)
    def _stats_pass():
        @pl.when(g == 0)
        def _init():
            s_acc[...] = jnp.zeros_like(s_acc)

        for bb in range(_BPS):
            s8 = [jnp.zeros((8, n_d), jnp.float32) for _ in range(n_k)]
            a1 = [jnp.zeros((8, n_d), jnp.float32) for _ in range(n_k)]
            a3 = [jnp.zeros((8, n_d), jnp.float32) for _ in range(n_k)]
            for c in range(n_ch):
                sl = slice(c * _CHUNK, (c + 1) * _CHUNK)
                xc = x_ref[bb, sl, :]
                es = []
                s = None
                for k in range(n_k):
                    d = xc - mean[k : k + 1, :]
                    e = jnp.exp2(d * d * nc2[k : k + 1, :])
                    es.append(e)
                    s = e if s is None else s + e
                r = 1.0 / (s + _EPS)
                for k in range(n_k):
                    tau = es[k] * r
                    t1 = tau * xc
                    t3 = t1 * t1 * tau
                    s8[k] = s8[k] + _vsum(tau)
                    a1[k] = a1[k] + _vsum(t1)
                    a3[k] = a3[k] + _vsum(t3)
            fin = lambda vs: jnp.concatenate(
                [jnp.sum(v, axis=0, keepdims=True) for v in vs], axis=0
            )
            s_acc[...] = s_acc[...] + fin(s8)
            a1_scr[g * _BPS + bb] = fin(a1)
            a3_scr[g * _BPS + bb] = fin(a3)

    @pl.when(ph == 1)
    def _output_pass():
        r_s = 1.0 / (s_acc[...] + _EPS)
        pri_scale = jnp.float32(1.0 / math.sqrt(1.0 + _EPS))
        inv_t = jnp.float32(1.0 / n_t)
        for bb in range(_BPS):
            a1 = a1_scr[g * _BPS + bb]
            a3 = a3_scr[g * _BPS + bb]
            e1 = r_s * a1 * inv_t
            e3 = (r_s * r_s * r_s) * a3 * inv_t
            w = jax.lax.rsqrt(e3 + _EPS) * pri_scale
            cw = e1 * w

            for c in range(n_ch):
                sl = slice(c * _CHUNK, (c + 1) * _CHUNK)
                xc = x_ref[bb, sl, :]
                es = []
                s = None
                for k in range(n_k):
                    d = xc - mean[k : k + 1, :]
                    e = jnp.exp2(d * d * nc2[k : k + 1, :])
                    es.append(e)
                    s = e if s is None else s + e
                r = 1.0 / (s + _EPS)
                p_sum = None
                q_sum = None
                for k in range(n_k):
                    pk = es[k] * w[k : k + 1, :]
                    qk = es[k] * cw[k : k + 1, :]
                    p_sum = pk if p_sum is None else p_sum + pk
                    q_sum = qk if q_sum is None else q_sum + qk
                out_ref[bb, sl, :] = r * (xc * p_sum - q_sum)


def kernel(x, mean, variance, prior):
    del prior
    n_b, n_t, n_d = x.shape
    n_k = mean.shape[0]
    return pl.pallas_call(
        _ucb_kernel,
        grid=(2, n_b // _BPS),
        in_specs=[
            pl.BlockSpec((_BPS, n_t, n_d), lambda ph, g: (g, 0, 0)),
            pl.BlockSpec((n_k, n_d), lambda ph, g: (0, 0)),
            pl.BlockSpec((n_k, n_d), lambda ph, g: (0, 0)),
        ],
        out_specs=pl.BlockSpec((_BPS, n_t, n_d), lambda ph, g: (g * ph, 0, 0)),
        out_shape=jax.ShapeDtypeStruct((n_b, n_t, n_d), jnp.float32),
        scratch_shapes=[
            pltpu.VMEM((n_k, n_d), jnp.float32),
            pltpu.VMEM((n_b, n_k, n_d), jnp.float32),
            pltpu.VMEM((n_b, n_k, n_d), jnp.float32),
        ],
        compiler_params=pltpu.CompilerParams(
            dimension_semantics=("arbitrary", "arbitrary"),
        ),
        name="ucb_norm",
    )(x, mean, variance)
